# baseline (speedup 1.0000x reference)
import functools

import jax
import jax.numpy as jnp
from jax import lax
from jax.experimental import pallas as pl
from jax.experimental.pallas import tpu as pltpu
from jax.experimental.pallas import tpu_sc as plsc

VOCAB = 1000000
EMBED = 64
BATCH = 16384
CTX = 20

NC = 2
NS = 16
NW = NC * NS
B_PER_W = BATCH // NW
N_CHUNK = 16
CB = B_PER_W // N_CHUNK
CROWS = CB * CTX
GPC = CROWS // 128

_TBN = 2048
_NBLK = (VOCAB + _TBN - 1) // _TBN
VOCAB_P = _NBLK * _TBN

_mesh = plsc.VectorSubcoreMesh(core_axis_name="c", subcore_axis_name="s")


def _worker_id():
    return lax.axis_index("s") * NC + lax.axis_index("c")


def _transpose_rm(table_t):
    def body(in_ref, out_ref):
        ii = lax.iota(jnp.int32, EMBED)
        ident = (ii[:, None] == ii[None, :]).astype(jnp.float32)
        dn = (((0,), (0,)), ((), ()))
        for h in range(2):
            xh = in_ref[:, h * (_TBN // 2):(h + 1) * (_TBN // 2)]
            out_ref[:, h * EMBED:(h + 1) * EMBED] = lax.dot_general(
                xh, ident, dn, precision=lax.Precision.HIGHEST)

    return pl.pallas_call(
        body,
        grid=(_NBLK,),
        in_specs=[pl.BlockSpec((EMBED, _TBN), lambda i: (0, i))],
        out_specs=pl.BlockSpec((_TBN // 2, 2 * EMBED), lambda i: (i, 0)),
        out_shape=jax.ShapeDtypeStruct((VOCAB_P // 2, 2 * EMBED),
                                       jnp.float32),
    )(table_t)


def _perm(w):
    h = _TBN // 2
    return (w // _TBN) * _TBN + 2 * (w % h) + (w % _TBN) // h


@functools.partial(
    pl.kernel,
    out_type=jax.ShapeDtypeStruct((BATCH // 2, 2 * EMBED), jnp.float32),
    mesh=_mesh,
    compiler_params=pltpu.CompilerParams(needs_layout_passes=False,
                                         use_tc_tiling_on_sc=True),
    scratch_types=[
        pltpu.VMEM((B_PER_W,), jnp.int32),
        pltpu.VMEM((8, EMBED, 128), jnp.float32),
        pltpu.VMEM((B_PER_W // 2, 2 * EMBED), jnp.float32),
        pltpu.SemaphoreType.DMA,
    ],
)
def _tgt_extract(tw_hbm, tt_hbm, out_hbm, widx, ring, rows, sem):
    wid = _worker_id()
    pltpu.sync_copy(tw_hbm.at[wid], widx)
    lane_iota = lax.iota(jnp.int32, 16)

    def start(w, slot):
        w128 = pl.multiple_of((w // 128) * 128, 128)
        pltpu.async_copy(tt_hbm.at[:, pl.ds(w128, 128)], ring.at[slot], sem)

    def finish(g, j, w, slot):
        pltpu.make_async_copy(tt_hbm.at[:, pl.ds(0, 128)], ring.at[slot],
                              sem).wait()
        colv = jnp.full((16,), w % 128, jnp.int32)
        for k in range(EMBED // 16):
            v = plsc.load_gather(ring.at[slot], [lane_iota + 16 * k, colv])
            rows[8 * g + j // 2, pl.ds((j % 2) * EMBED + 16 * k, 16)] = v

    def step(g, carry):
        wv = widx[pl.ds(g * 16, 16)]
        for j in range(8):
            start(wv[j], j)
        for j in range(8):
            finish(g, j, wv[j], j)
            start(wv[j + 8], j)
        for j in range(8):
            finish(g, j + 8, wv[j + 8], j)
        return carry

    lax.fori_loop(0, B_PER_W // 16, step, 0, unroll=False)
    pltpu.sync_copy(rows, out_hbm.at[pl.ds(wid * (B_PER_W // 2),
                                           B_PER_W // 2)])


@functools.partial(
    pl.kernel,
    out_type=jax.ShapeDtypeStruct((BATCH * CTX,), jnp.float32),
    mesh=_mesh,
    compiler_params=pltpu.CompilerParams(needs_layout_passes=False,
                                         use_tc_tiling_on_sc=False),
    scratch_types=[
        pltpu.VMEM((CTX * B_PER_W // 128, 128), jnp.int32),
        pltpu.VMEM((B_PER_W, EMBED), jnp.float32),
        pltpu.VMEM((CROWS, EMBED), jnp.float32),
        pltpu.VMEM((B_PER_W * CTX,), jnp.float32),
        pltpu.SemaphoreType.DMA,
    ],
)
def _skipgram_sc(cw_hbm, te_hbm, ct_hbm, out_hbm,
                 cidx, tgt_buf, ctx_buf, out_buf, sem):
    wid = _worker_id()

    pltpu.sync_copy(cw_hbm.at[wid], cidx)
    pltpu.sync_copy(te_hbm.at[pl.ds(wid * B_PER_W, B_PER_W)], tgt_buf)

    def chunk_body(chunk, carry):
        for j in range(GPC):
            pltpu.async_copy(ct_hbm.at[cidx.at[chunk * GPC + j]],
                             ctx_buf.at[pl.ds(j * 128, 128)], sem)
        for j in range(GPC):
            pltpu.make_async_copy(ct_hbm.at[cidx.at[chunk * GPC + j]],
                                  ctx_buf.at[pl.ds(j * 128, 128)], sem).wait()

        lane_iota = lax.iota(jnp.int32, 16)

        def group_body(g, carry2):
            accs = [jnp.zeros((16,), jnp.float32) for _ in range(5)]
            for bl in range(4):
                b = chunk * CB + g * 4 + bl
                t = [tgt_buf[b, pl.ds(k * 16, 16)]
                     for k in range(EMBED // 16)]
                for c in range(CTX):
                    r = g * (4 * CTX) + bl * CTX + c
                    p = t[0] * ctx_buf[r, pl.ds(0, 16)]
                    for k in range(1, EMBED // 16):
                        p = p + t[k] * ctx_buf[r, pl.ds(k * 16, 16)]
                    s = jnp.sum(p)
                    v, lane = divmod(bl * CTX + c, 16)
                    accs[v] = jnp.where(lane_iota == lane, s, accs[v])
            base = chunk * (CB * CTX) + g * 80
            for v in range(5):
                out_buf[pl.ds(base + v * 16, 16)] = accs[v]
            return carry2

        lax.fori_loop(0, CB // 4, group_body, 0, unroll=False)
        return carry

    lax.fori_loop(0, N_CHUNK, chunk_body, 0, unroll=False)

    pltpu.sync_copy(out_buf, out_hbm.at[pl.ds(wid * B_PER_W * CTX,
                                              B_PER_W * CTX)])


def kernel(target_words, context_words, target_table, context_table):
    tw = target_words.reshape(NW, B_PER_W)
    cwp = _perm(context_words).reshape(NW, CTX * B_PER_W // 128, 128)
    ct_rm = _transpose_rm(context_table.T).reshape(VOCAB_P, EMBED)
    tembeds = _tgt_extract(tw, target_table.T).reshape(BATCH, EMBED)
    out = _skipgram_sc(cwp, tembeds, ct_rm)
    return out.reshape(BATCH, CTX)


# speedup vs baseline: 1.2279x; 1.2279x over previous
import functools

import jax
import jax.numpy as jnp
from jax import lax
from jax.experimental import pallas as pl
from jax.experimental.pallas import tpu as pltpu
from jax.experimental.pallas import tpu_sc as plsc

VOCAB = 1000000
EMBED = 64
BATCH = 16384
CTX = 20

NC = 2
NS = 16
NW = NC * NS
B_PER_W = BATCH // NW
N_CHUNK = 16
CB = B_PER_W // N_CHUNK
CROWS = CB * CTX
GPC = CROWS // 128

_TBN = 2048
_NBLK = (VOCAB + _TBN - 1) // _TBN
VOCAB_P = _NBLK * _TBN

_mesh = plsc.VectorSubcoreMesh(core_axis_name="c", subcore_axis_name="s")


def _worker_id():
    return lax.axis_index("s") * NC + lax.axis_index("c")


def _transpose_rm(table_t):
    def body(in_ref, out_ref):
        ii = lax.iota(jnp.int32, EMBED)
        ident = (ii[:, None] == ii[None, :]).astype(jnp.float32)
        dn = (((0,), (0,)), ((), ()))
        for h in range(2):
            xh = in_ref[:, h * (_TBN // 2):(h + 1) * (_TBN // 2)]
            out_ref[:, h * EMBED:(h + 1) * EMBED] = lax.dot_general(
                xh, ident, dn, precision=lax.Precision.DEFAULT)

    return pl.pallas_call(
        body,
        grid=(_NBLK,),
        in_specs=[pl.BlockSpec((EMBED, _TBN), lambda i: (0, i))],
        out_specs=pl.BlockSpec((_TBN // 2, 2 * EMBED), lambda i: (i, 0)),
        out_shape=jax.ShapeDtypeStruct((VOCAB_P // 2, 2 * EMBED),
                                       jnp.float32),
    )(table_t)


def _perm(w):
    h = _TBN // 2
    return (w // _TBN) * _TBN + 2 * (w % h) + (w % _TBN) // h


@functools.partial(
    pl.kernel,
    out_type=jax.ShapeDtypeStruct((BATCH // 2, 2 * EMBED), jnp.float32),
    mesh=_mesh,
    compiler_params=pltpu.CompilerParams(needs_layout_passes=False,
                                         use_tc_tiling_on_sc=True),
    scratch_types=[
        pltpu.VMEM((B_PER_W,), jnp.int32),
        pltpu.VMEM((8, EMBED, 128), jnp.float32),
        pltpu.VMEM((B_PER_W // 2, 2 * EMBED), jnp.float32),
        pltpu.SemaphoreType.DMA,
    ],
)
def _tgt_extract(tw_hbm, tt_hbm, out_hbm, widx, ring, rows, sem):
    wid = _worker_id()
    pltpu.sync_copy(tw_hbm.at[wid], widx)
    lane_iota = lax.iota(jnp.int32, 16)

    def start(w, slot):
        w128 = pl.multiple_of((w // 128) * 128, 128)
        pltpu.async_copy(tt_hbm.at[:, pl.ds(w128, 128)], ring.at[slot], sem)

    def finish(g, j, w, slot):
        pltpu.make_async_copy(tt_hbm.at[:, pl.ds(0, 128)], ring.at[slot],
                              sem).wait()
        colv = jnp.full((16,), w % 128, jnp.int32)
        for k in range(EMBED // 16):
            v = plsc.load_gather(ring.at[slot], [lane_iota + 16 * k, colv])
            rows[8 * g + j // 2, pl.ds((j % 2) * EMBED + 16 * k, 16)] = v

    def step(g, carry):
        wv = widx[pl.ds(g * 16, 16)]
        for j in range(8):
            start(wv[j], j)
        for j in range(8):
            finish(g, j, wv[j], j)
            start(wv[j + 8], j)
        for j in range(8):
            finish(g, j + 8, wv[j + 8], j)
        return carry

    lax.fori_loop(0, B_PER_W // 16, step, 0, unroll=False)
    pltpu.sync_copy(rows, out_hbm.at[pl.ds(wid * (B_PER_W // 2),
                                           B_PER_W // 2)])


@functools.partial(
    pl.kernel,
    out_type=jax.ShapeDtypeStruct((BATCH * CTX,), jnp.float32),
    mesh=_mesh,
    compiler_params=pltpu.CompilerParams(needs_layout_passes=False,
                                         use_tc_tiling_on_sc=False),
    scratch_types=[
        pltpu.VMEM((CTX * B_PER_W // 128, 128), jnp.int32),
        pltpu.VMEM((B_PER_W, EMBED), jnp.float32),
        pltpu.VMEM((CROWS, EMBED), jnp.float32),
        pltpu.VMEM((B_PER_W * CTX,), jnp.float32),
        pltpu.SemaphoreType.DMA,
    ],
)
def _skipgram_sc(cw_hbm, te_hbm, ct_hbm, out_hbm,
                 cidx, tgt_buf, ctx_buf, out_buf, sem):
    wid = _worker_id()

    pltpu.sync_copy(cw_hbm.at[wid], cidx)
    pltpu.sync_copy(te_hbm.at[pl.ds(wid * B_PER_W, B_PER_W)], tgt_buf)

    def chunk_body(chunk, carry):
        for j in range(GPC):
            pltpu.async_copy(ct_hbm.at[cidx.at[chunk * GPC + j]],
                             ctx_buf.at[pl.ds(j * 128, 128)], sem)
        for j in range(GPC):
            pltpu.make_async_copy(ct_hbm.at[cidx.at[chunk * GPC + j]],
                                  ctx_buf.at[pl.ds(j * 128, 128)], sem).wait()

        lane_iota = lax.iota(jnp.int32, 16)

        def group_body(g, carry2):
            accs = [jnp.zeros((16,), jnp.float32) for _ in range(5)]
            for bl in range(4):
                b = chunk * CB + g * 4 + bl
                t = [tgt_buf[b, pl.ds(k * 16, 16)]
                     for k in range(EMBED // 16)]
                for c in range(CTX):
                    r = g * (4 * CTX) + bl * CTX + c
                    p = t[0] * ctx_buf[r, pl.ds(0, 16)]
                    for k in range(1, EMBED // 16):
                        p = p + t[k] * ctx_buf[r, pl.ds(k * 16, 16)]
                    s = jnp.sum(p)
                    v, lane = divmod(bl * CTX + c, 16)
                    accs[v] = jnp.where(lane_iota == lane, s, accs[v])
            base = chunk * (CB * CTX) + g * 80
            for v in range(5):
                out_buf[pl.ds(base + v * 16, 16)] = accs[v]
            return carry2

        lax.fori_loop(0, CB // 4, group_body, 0, unroll=False)
        return carry

    lax.fori_loop(0, N_CHUNK, chunk_body, 0, unroll=False)

    pltpu.sync_copy(out_buf, out_hbm.at[pl.ds(wid * B_PER_W * CTX,
                                              B_PER_W * CTX)])


def kernel(target_words, context_words, target_table, context_table):
    tw = target_words.reshape(NW, B_PER_W)
    cwp = _perm(context_words).reshape(NW, CTX * B_PER_W // 128, 128)
    ct_rm = _transpose_rm(context_table.T).reshape(VOCAB_P, EMBED)
    tembeds = _tgt_extract(tw, target_table.T).reshape(BATCH, EMBED)
    out = _skipgram_sc(cwp, tembeds, ct_rm)
    return out.reshape(BATCH, CTX)


# speedup vs baseline: 1.7076x; 1.3907x over previous
import functools

import jax
import jax.numpy as jnp
from jax import lax
from jax.experimental import pallas as pl
from jax.experimental.pallas import tpu as pltpu
from jax.experimental.pallas import tpu_sc as plsc

VOCAB = 1000000
EMBED = 64
BATCH = 16384
CTX = 20

NC = 2
NS = 16
NW = NC * NS
B_PER_W = BATCH // NW
N_CHUNK = 16
CB = B_PER_W // N_CHUNK
CROWS = CB * CTX
GPC = CROWS // 128

_TBN = 8192
_NBLK = (VOCAB + _TBN - 1) // _TBN
VOCAB_P = _NBLK * _TBN

_mesh = plsc.VectorSubcoreMesh(core_axis_name="c", subcore_axis_name="s")


def _worker_id():
    return lax.axis_index("s") * NC + lax.axis_index("c")


def _transpose_rm(table_t):
    def body(in_ref, out_ref):
        out_ref[:, 0:EMBED] = in_ref[:, 0:_TBN // 2].T
        out_ref[:, EMBED:2 * EMBED] = in_ref[:, _TBN // 2:_TBN].T

    return pl.pallas_call(
        body,
        grid=(_NBLK,),
        in_specs=[pl.BlockSpec((EMBED, _TBN), lambda i: (0, i))],
        out_specs=pl.BlockSpec((_TBN // 2, 2 * EMBED), lambda i: (i, 0)),
        out_shape=jax.ShapeDtypeStruct((VOCAB_P // 2, 2 * EMBED),
                                       jnp.float32),
    )(table_t)


def _perm(w):
    h = _TBN // 2
    return (w // _TBN) * _TBN + 2 * (w % h) + (w % _TBN) // h


@functools.partial(
    pl.kernel,
    out_type=jax.ShapeDtypeStruct((BATCH // 2, 2 * EMBED), jnp.float32),
    mesh=_mesh,
    compiler_params=pltpu.CompilerParams(needs_layout_passes=False,
                                         use_tc_tiling_on_sc=True),
    scratch_types=[
        pltpu.VMEM((B_PER_W,), jnp.int32),
        pltpu.VMEM((8, EMBED, 128), jnp.float32),
        pltpu.VMEM((B_PER_W // 2, 2 * EMBED), jnp.float32),
        pltpu.SemaphoreType.DMA,
    ],
)
def _tgt_extract(tw_hbm, tt_hbm, out_hbm, widx, ring, rows, sem):
    wid = _worker_id()
    pltpu.sync_copy(tw_hbm.at[wid], widx)
    lane_iota = lax.iota(jnp.int32, 16)

    def start(w, slot):
        w128 = pl.multiple_of((w // 128) * 128, 128)
        pltpu.async_copy(tt_hbm.at[:, pl.ds(w128, 128)], ring.at[slot], sem)

    def finish(g, j, w, slot):
        pltpu.make_async_copy(tt_hbm.at[:, pl.ds(0, 128)], ring.at[slot],
                              sem).wait()
        colv = jnp.full((16,), w % 128, jnp.int32)
        for k in range(EMBED // 16):
            v = plsc.load_gather(ring.at[slot], [lane_iota + 16 * k, colv])
            rows[8 * g + j // 2, pl.ds((j % 2) * EMBED + 16 * k, 16)] = v

    def step(g, carry):
        wv = widx[pl.ds(g * 16, 16)]
        for j in range(8):
            start(wv[j], j)
        for j in range(8):
            finish(g, j, wv[j], j)
            start(wv[j + 8], j)
        for j in range(8):
            finish(g, j + 8, wv[j + 8], j)
        return carry

    lax.fori_loop(0, B_PER_W // 16, step, 0, unroll=False)
    pltpu.sync_copy(rows, out_hbm.at[pl.ds(wid * (B_PER_W // 2),
                                           B_PER_W // 2)])


@functools.partial(
    pl.kernel,
    out_type=jax.ShapeDtypeStruct((BATCH * CTX,), jnp.float32),
    mesh=_mesh,
    compiler_params=pltpu.CompilerParams(needs_layout_passes=False,
                                         use_tc_tiling_on_sc=False),
    scratch_types=[
        pltpu.VMEM((CTX * B_PER_W // 128, 128), jnp.int32),
        pltpu.VMEM((B_PER_W, EMBED), jnp.float32),
        pltpu.VMEM((CROWS, EMBED), jnp.float32),
        pltpu.VMEM((B_PER_W * CTX,), jnp.float32),
        pltpu.SemaphoreType.DMA,
    ],
)
def _skipgram_sc(cw_hbm, te_hbm, ct_hbm, out_hbm,
                 cidx, tgt_buf, ctx_buf, out_buf, sem):
    wid = _worker_id()

    pltpu.sync_copy(cw_hbm.at[wid], cidx)
    pltpu.sync_copy(te_hbm.at[pl.ds(wid * B_PER_W, B_PER_W)], tgt_buf)

    def chunk_body(chunk, carry):
        for j in range(GPC):
            pltpu.async_copy(ct_hbm.at[cidx.at[chunk * GPC + j]],
                             ctx_buf.at[pl.ds(j * 128, 128)], sem)
        for j in range(GPC):
            pltpu.make_async_copy(ct_hbm.at[cidx.at[chunk * GPC + j]],
                                  ctx_buf.at[pl.ds(j * 128, 128)], sem).wait()

        lane_iota = lax.iota(jnp.int32, 16)

        def group_body(g, carry2):
            accs = [jnp.zeros((16,), jnp.float32) for _ in range(5)]
            for bl in range(4):
                b = chunk * CB + g * 4 + bl
                t = [tgt_buf[b, pl.ds(k * 16, 16)]
                     for k in range(EMBED // 16)]
                for c in range(CTX):
                    r = g * (4 * CTX) + bl * CTX + c
                    p = t[0] * ctx_buf[r, pl.ds(0, 16)]
                    for k in range(1, EMBED // 16):
                        p = p + t[k] * ctx_buf[r, pl.ds(k * 16, 16)]
                    s = jnp.sum(p)
                    v, lane = divmod(bl * CTX + c, 16)
                    accs[v] = jnp.where(lane_iota == lane, s, accs[v])
            base = chunk * (CB * CTX) + g * 80
            for v in range(5):
                out_buf[pl.ds(base + v * 16, 16)] = accs[v]
            return carry2

        lax.fori_loop(0, CB // 4, group_body, 0, unroll=False)
        return carry

    lax.fori_loop(0, N_CHUNK, chunk_body, 0, unroll=False)

    pltpu.sync_copy(out_buf, out_hbm.at[pl.ds(wid * B_PER_W * CTX,
                                              B_PER_W * CTX)])


def kernel(target_words, context_words, target_table, context_table):
    tw = target_words.reshape(NW, B_PER_W)
    cwp = _perm(context_words).reshape(NW, CTX * B_PER_W // 128, 128)
    ct_rm = _transpose_rm(context_table.T).reshape(VOCAB_P, EMBED)
    tembeds = _tgt_extract(tw, target_table.T).reshape(BATCH, EMBED)
    out = _skipgram_sc(cwp, tembeds, ct_rm)
    return out.reshape(BATCH, CTX)
